# baseline (speedup 1.0000x reference)
import functools

import jax
import jax.numpy as jnp
from jax import lax
from jax.experimental import pallas as pl
from jax.experimental.pallas import tpu as pltpu
from jax.experimental.pallas import tpu_sc as plsc

_B, _C, _T, _D = 8, 64, 1024, 128
_Q = _T // _C
_NW = 32
_PAIRS = _C * _Q
_PPW = _PAIRS // _NW
_ROWS = _C
_L = 16


def _chunk_base(p0, k):
    j = k // _B
    bi = k % _B
    p = p0 + j
    ch = p // _Q
    q = p % _Q
    return j, bi * (_C * _T) + ch * _T + q * _ROWS


def _sc_body(x_hbm, tt_hbm, ct_hbm, o_hbm, ct_v, tt_v,
             in0, in1, out0, out1, isem0, isem1, osem0, osem1):
    cid = lax.axis_index("c")
    sid = lax.axis_index("s")
    wid = sid * 2 + cid
    p0 = wid * _PPW
    pltpu.sync_copy(ct_hbm, ct_v)
    pltpu.sync_copy(tt_hbm.at[pl.ds(p0, _PPW)], tt_v)

    ins = (in0, in1)
    outs = (out0, out1)
    isems = (isem0, isem1)
    osems = (osem0, osem1)
    n_chunks = _PPW * _B

    for s in range(2):
        _, base = _chunk_base(p0, s)
        pltpu.async_copy(x_hbm.at[pl.ds(base, _ROWS)], ins[s], isems[s])

    def step(k0, carry):
        for s in range(2):
            k = k0 * 2 + s
            j, base = _chunk_base(p0, k)
            inb, outb = ins[s], outs[s]
            pltpu.make_async_copy(x_hbm.at[pl.ds(base, _ROWS)], inb,
                                  isems[s]).wait()

            @pl.when(k >= 2)
            def _():
                pltpu.make_async_copy(outb, o_hbm.at[pl.ds(base, _ROWS)],
                                      osems[s]).wait()

            tts = tuple(tt_v[j, pl.ds(g * _L, _L)] for g in range(_D // _L))

            @plsc.parallel_loop(0, _ROWS, step=1, unroll=4)
            def _(r):
                for g in range(_D // _L):
                    sl = pl.ds(g * _L, _L)
                    outb[r, sl] = inb[r, sl] + ct_v[r, sl] + tts[g]

            @pl.when(k + 2 < n_chunks)
            def _():
                _, nbase = _chunk_base(p0, k + 2)
                pltpu.async_copy(x_hbm.at[pl.ds(nbase, _ROWS)], inb, isems[s])

            pltpu.async_copy(outb, o_hbm.at[pl.ds(base, _ROWS)], osems[s])
        return carry

    lax.fori_loop(0, n_chunks // 2, step, 0)
    for s in range(2):
        pltpu.make_async_copy(outs[s], o_hbm.at[pl.ds(0, _ROWS)],
                              osems[s]).wait()


def kernel(x, time_table, channel_table):
    b, c, T, D = x.shape
    assert (b, c, T, D) == (_B, _C, _T, _D)
    mesh = plsc.VectorSubcoreMesh(
        core_axis_name="c", subcore_axis_name="s", num_cores=2, num_subcores=16
    )
    f = pl.kernel(
        _sc_body,
        out_type=jax.ShapeDtypeStruct((b * c * T, D), x.dtype),
        mesh=mesh,
        scratch_types=[
            pltpu.VMEM((_ROWS, _D), jnp.float32),
            pltpu.VMEM((_PPW, _D), jnp.float32),
            pltpu.VMEM((_ROWS, _D), jnp.float32),
            pltpu.VMEM((_ROWS, _D), jnp.float32),
            pltpu.VMEM((_ROWS, _D), jnp.float32),
            pltpu.VMEM((_ROWS, _D), jnp.float32),
            pltpu.SemaphoreType.DMA,
            pltpu.SemaphoreType.DMA,
            pltpu.SemaphoreType.DMA,
            pltpu.SemaphoreType.DMA,
        ],
    )
    out = f(x.reshape(b * c * T, D), time_table, channel_table)
    return out.reshape(b, c * T, D)


# speedup vs baseline: 1.2023x; 1.2023x over previous
import functools

import jax
import jax.numpy as jnp
from jax import lax
from jax.experimental import pallas as pl
from jax.experimental.pallas import tpu as pltpu
from jax.experimental.pallas import tpu_sc as plsc

_B, _C, _T, _D = 8, 64, 1024, 128
_Q = _T // _C
_NW = 32
_PAIRS = _C * _Q
_PPW = _PAIRS // _NW
_ROWS = _C
_CR = 2 * _ROWS
_L = 16


def _chunk_base(p0, k):
    j = 2 * (k // _B)
    bi = k % _B
    p = p0 + j
    ch = p // _Q
    q = p % _Q
    return j, bi * (_C * _T) + ch * _T + q * _ROWS


def _sc_body(x_hbm, tt_hbm, ct_hbm, o_hbm, ct_v, tt_v,
             in0, in1, out0, out1, isem0, isem1, osem0, osem1):
    cid = lax.axis_index("c")
    sid = lax.axis_index("s")
    wid = sid * 2 + cid
    p0 = wid * _PPW
    pltpu.sync_copy(ct_hbm, ct_v)
    pltpu.sync_copy(tt_hbm.at[pl.ds(p0, _PPW)], tt_v)

    ins = (in0, in1)
    outs = (out0, out1)
    isems = (isem0, isem1)
    osems = (osem0, osem1)
    n_chunks = _PPW * _B // 2

    for s in range(2):
        _, base = _chunk_base(p0, s)
        pltpu.async_copy(x_hbm.at[pl.ds(base, _CR)], ins[s], isems[s])

    def step(k0, carry):
        for s in range(2):
            k = k0 * 2 + s
            j, base = _chunk_base(p0, k)
            inb, outb = ins[s], outs[s]
            pltpu.make_async_copy(x_hbm.at[pl.ds(base, _CR)], inb,
                                  isems[s]).wait()

            @pl.when(k >= 2)
            def _():
                pltpu.make_async_copy(outb, o_hbm.at[pl.ds(base, _CR)],
                                      osems[s]).wait()

            for h in range(2):
                tts = tuple(tt_v[j + h, pl.ds(g * _L, _L)]
                            for g in range(_D // _L))

                @plsc.parallel_loop(0, _ROWS, step=1, unroll=4)
                def _(r):
                    for g in range(_D // _L):
                        sl = pl.ds(g * _L, _L)
                        outb[h * _ROWS + r, sl] = (
                            inb[h * _ROWS + r, sl] + ct_v[r, sl] + tts[g])

            @pl.when(k + 2 < n_chunks)
            def _():
                _, nbase = _chunk_base(p0, k + 2)
                pltpu.async_copy(x_hbm.at[pl.ds(nbase, _CR)], inb, isems[s])

            pltpu.async_copy(outb, o_hbm.at[pl.ds(base, _CR)], osems[s])
        return carry

    lax.fori_loop(0, n_chunks // 2, step, 0)
    for s in range(2):
        pltpu.make_async_copy(outs[s], o_hbm.at[pl.ds(0, _CR)],
                              osems[s]).wait()


def kernel(x, time_table, channel_table):
    b, c, T, D = x.shape
    assert (b, c, T, D) == (_B, _C, _T, _D)
    mesh = plsc.VectorSubcoreMesh(
        core_axis_name="c", subcore_axis_name="s", num_cores=2, num_subcores=16
    )
    f = pl.kernel(
        _sc_body,
        out_type=jax.ShapeDtypeStruct((b * c * T, D), x.dtype),
        mesh=mesh,
        scratch_types=[
            pltpu.VMEM((_ROWS, _D), jnp.float32),
            pltpu.VMEM((_PPW, _D), jnp.float32),
            pltpu.VMEM((_CR, _D), jnp.float32),
            pltpu.VMEM((_CR, _D), jnp.float32),
            pltpu.VMEM((_CR, _D), jnp.float32),
            pltpu.VMEM((_CR, _D), jnp.float32),
            pltpu.SemaphoreType.DMA,
            pltpu.SemaphoreType.DMA,
            pltpu.SemaphoreType.DMA,
            pltpu.SemaphoreType.DMA,
        ],
    )
    out = f(x.reshape(b * c * T, D), time_table, channel_table)
    return out.reshape(b, c * T, D)


# speedup vs baseline: 1.2050x; 1.0023x over previous
import jax
import jax.numpy as jnp
from jax import lax
from jax.experimental import pallas as pl
from jax.experimental.pallas import tpu as pltpu
from jax.experimental.pallas import tpu_sc as plsc

_B, _C, _T, _D = 8, 64, 1024, 128
_Q = _T // _C
_NW = 32
_PAIRS = _C * _Q
_PPW = _PAIRS // _NW
_ROWS = _C
_CR = 2 * _ROWS
_L = 16


def _chunk_base(p0, k):
    j = 2 * (k // _B)
    bi = k % _B
    p = p0 + j
    ch = p // _Q
    q = p % _Q
    return j, bi * (_C * _T) + ch * _T + q * _ROWS


def _sc_body(x_hbm, tt_hbm, ct_hbm, o_hbm, ct_v, tt_v,
             in0, in1, out0, out1, isem0, isem1, osem0, osem1):
    cid = lax.axis_index("c")
    sid = lax.axis_index("s")
    wid = sid * 2 + cid
    p0 = wid * _PPW
    pltpu.sync_copy(ct_hbm, ct_v)
    pltpu.sync_copy(tt_hbm.at[pl.ds(p0, _PPW)], tt_v)

    ins = (in0, in1)
    outs = (out0, out1)
    isems = (isem0, isem1)
    osems = (osem0, osem1)
    n_chunks = _PPW * _B // 2

    for s in range(2):
        _, base = _chunk_base(p0, s)
        pltpu.async_copy(x_hbm.at[pl.ds(base, _CR)], ins[s], isems[s])

    def step(k0, carry):
        for s in range(2):
            k = k0 * 2 + s
            j, base = _chunk_base(p0, k)
            inb, outb = ins[s], outs[s]
            pltpu.make_async_copy(x_hbm.at[pl.ds(base, _CR)], inb,
                                  isems[s]).wait()

            @pl.when(k >= 2)
            def _():
                pltpu.make_async_copy(outb, o_hbm.at[pl.ds(base, _CR)],
                                      osems[s]).wait()

            for h in range(2):
                tts = tuple(tt_v[j + h, pl.ds(g * _L, _L)]
                            for g in range(_D // _L))

                @plsc.parallel_loop(0, _ROWS, step=1, unroll=4)
                def _(r):
                    for g in range(_D // _L):
                        sl = pl.ds(g * _L, _L)
                        outb[h * _ROWS + r, sl] = (
                            inb[h * _ROWS + r, sl] + ct_v[r, sl] + tts[g])

            @pl.when(k + 2 < n_chunks)
            def _():
                _, nbase = _chunk_base(p0, k + 2)
                pltpu.async_copy(x_hbm.at[pl.ds(nbase, _CR)], inb, isems[s])

            pltpu.async_copy(outb, o_hbm.at[pl.ds(base, _CR)], osems[s])
        return carry

    lax.fori_loop(0, n_chunks // 2, step, 0)
    for s in range(2):
        pltpu.make_async_copy(outs[s], o_hbm.at[pl.ds(0, _CR)],
                              osems[s]).wait()


def kernel(x, time_table, channel_table):
    b, c, T, D = x.shape
    assert (b, c, T, D) == (_B, _C, _T, _D)
    mesh = plsc.VectorSubcoreMesh(
        core_axis_name="c", subcore_axis_name="s", num_cores=2, num_subcores=16
    )
    f = pl.kernel(
        _sc_body,
        out_type=jax.ShapeDtypeStruct((b * c * T, D), x.dtype),
        mesh=mesh,
        scratch_types=[
            pltpu.VMEM((_ROWS, _D), jnp.float32),
            pltpu.VMEM((_PPW, _D), jnp.float32),
            pltpu.VMEM((_CR, _D), jnp.float32),
            pltpu.VMEM((_CR, _D), jnp.float32),
            pltpu.VMEM((_CR, _D), jnp.float32),
            pltpu.VMEM((_CR, _D), jnp.float32),
            pltpu.SemaphoreType.DMA,
            pltpu.SemaphoreType.DMA,
            pltpu.SemaphoreType.DMA,
            pltpu.SemaphoreType.DMA,
        ],
    )
    out = f(x.reshape(b * c * T, D), time_table, channel_table)
    return out.reshape(b, c * T, D)
